# Initial kernel scaffold; baseline (speedup 1.0000x reference)
import functools

import jax
import jax.numpy as jnp
from jax import lax
from jax.experimental import pallas as pl
from jax.experimental.pallas import tpu as pltpu
from jax.experimental.pallas import tpu_sc as plsc

N = 10000
E = 160000
D = 256
DE = 16
H = 256

NC = 2
NS = 16
NW = NC * NS
CHUNK = 128
KCH = -(-E // (NW * CHUNK))
EPT = KCH * CHUNK
E_PAD = NW * EPT

ROWS_PER_TILE = N // NS


def _sc_scatter_kernel(idx_hbm, attr_hbm, zero_hbm, out_hbm, idx_v, attr_v, acc):
    c = lax.axis_index("c")
    s = lax.axis_index("s")
    wid = c * NS + s

    pltpu.sync_copy(zero_hbm.at[pl.ds(s * ROWS_PER_TILE, ROWS_PER_TILE)],
                    acc.at[pl.ds(s * ROWS_PER_TILE, ROWS_PER_TILE)])

    pltpu.sync_copy(idx_hbm.at[wid], idx_v)
    pltpu.sync_copy(attr_hbm.at[pl.ds(wid * EPT, EPT)], attr_v)

    plsc.subcore_barrier()

    @pl.loop(0, KCH)
    def _scatter(j):
        pltpu.sync_copy(attr_v.at[pl.ds(j * CHUNK, CHUNK)],
                        acc.at[idx_v.at[j]], add=True)

    plsc.subcore_barrier()

    pltpu.sync_copy(acc.at[pl.ds(s * ROWS_PER_TILE, ROWS_PER_TILE)],
                    out_hbm.at[c, pl.ds(s * ROWS_PER_TILE, ROWS_PER_TILE)])


def _sc_scatter(idx3, attr_pad, zeros_n):
    mesh = plsc.VectorSubcoreMesh(core_axis_name="c", subcore_axis_name="s")
    return pl.kernel(
        _sc_scatter_kernel,
        out_type=jax.ShapeDtypeStruct((NC, N, DE), jnp.float32),
        mesh=mesh,
        scratch_types=[
            pltpu.VMEM((KCH, CHUNK), jnp.int32),
            pltpu.VMEM((EPT, DE), jnp.float32),
            pltpu.VMEM_SHARED((N, DE), jnp.float32),
        ],
    )(idx3, attr_pad, zeros_n)


def _mlp_kernel(x_ref, ea0_ref, ea1_ref, w1x_ref, w1e_ref, b1_ref,
                w2_ref, b2_ref, w3_ref, b3_ref, out_ref):
    ea = ea0_ref[...] + ea1_ref[...]
    h = (jnp.dot(x_ref[...], w1x_ref[...], preferred_element_type=jnp.float32)
         + jnp.dot(ea, w1e_ref[...], preferred_element_type=jnp.float32)
         + b1_ref[...])
    h = jnp.where(h >= 0, h, 0.01 * h)
    h = jnp.dot(h, w2_ref[...], preferred_element_type=jnp.float32) + b2_ref[...]
    h = jnp.where(h >= 0, h, 0.01 * h)
    h = jnp.dot(h, w3_ref[...], preferred_element_type=jnp.float32) + b3_ref[...]
    out_ref[...] = h


def _mlp(x, ea0, ea1, w1x, w1e, b1, w2, b2, w3, b3):
    BLK = 1000
    grid = (N // BLK,)
    full = lambda shape: pl.BlockSpec(shape, lambda i: (0, 0))
    return pl.pallas_call(
        _mlp_kernel,
        grid=grid,
        in_specs=[
            pl.BlockSpec((BLK, D), lambda i: (i, 0)),
            pl.BlockSpec((BLK, DE), lambda i: (i, 0)),
            pl.BlockSpec((BLK, DE), lambda i: (i, 0)),
            full((D, H)),
            full((DE, H)),
            full((1, H)),
            full((H, H)),
            full((1, H)),
            full((H, H)),
            full((1, H)),
        ],
        out_specs=pl.BlockSpec((BLK, H), lambda i: (i, 0)),
        out_shape=jax.ShapeDtypeStruct((N, H), jnp.float32),
    )(x, ea0, ea1, w1x, w1e, b1, w2, b2, w3, b3)


def kernel(x, edge_index, edge_attr, u, batch, W1, b1, W2, b2, W3, b3):
    row = edge_index[0]
    pad = E_PAD - E
    row_pad = jnp.concatenate([row, jnp.zeros((pad,), jnp.int32)])
    attr_pad = jnp.concatenate(
        [edge_attr, jnp.zeros((pad, DE), jnp.float32)], axis=0)
    idx3 = row_pad.reshape(NW, KCH, CHUNK)
    zeros_n = jnp.zeros((N, DE), jnp.float32)

    ea2 = _sc_scatter(idx3, attr_pad, zeros_n)

    return _mlp(x, ea2[0], ea2[1], W1[:D], W1[D:], b1[None, :],
                W2, b2[None, :], W3, b3[None, :])


# speedup vs baseline: 3.0460x; 3.0460x over previous
import functools

import jax
import jax.numpy as jnp
from jax import lax
from jax.experimental import pallas as pl
from jax.experimental.pallas import tpu as pltpu
from jax.experimental.pallas import tpu_sc as plsc

N = 10000
E = 160000
D = 256
DE = 16
H = 256

NC = 2
NS = 16
NW = NC * NS
CHUNK = 128
KCH = -(-E // (NW * CHUNK))
EPT = KCH * CHUNK
E_PAD = NW * EPT

N_PAD = 10240
ROWS_PER_TILE = N_PAD // NS


def _sc_scatter_kernel(idx_hbm, attr_hbm, zero_hbm, out_hbm, idx_v, attr_v, acc):
    c = lax.axis_index("c")
    s = lax.axis_index("s")
    wid = c * NS + s

    pltpu.sync_copy(zero_hbm.at[pl.ds(s * ROWS_PER_TILE, ROWS_PER_TILE)],
                    acc.at[pl.ds(s * ROWS_PER_TILE, ROWS_PER_TILE)])

    pltpu.sync_copy(idx_hbm.at[wid], idx_v)
    pltpu.sync_copy(attr_hbm.at[pl.ds(wid * EPT, EPT)], attr_v)

    plsc.subcore_barrier()

    @pl.loop(0, KCH)
    def _scatter(j):
        pltpu.sync_copy(attr_v.at[pl.ds(j * CHUNK, CHUNK)],
                        acc.at[idx_v.at[j]], add=True)

    plsc.subcore_barrier()

    pltpu.sync_copy(acc.at[pl.ds(s * ROWS_PER_TILE, ROWS_PER_TILE)],
                    out_hbm.at[c, pl.ds(s * ROWS_PER_TILE, ROWS_PER_TILE)])


def _sc_scatter(idx3, attr_pad, zeros_n):
    mesh = plsc.VectorSubcoreMesh(core_axis_name="c", subcore_axis_name="s")
    return pl.kernel(
        _sc_scatter_kernel,
        out_type=jax.ShapeDtypeStruct((NC, N_PAD, DE), jnp.float32),
        mesh=mesh,
        scratch_types=[
            pltpu.VMEM((KCH, CHUNK), jnp.int32),
            pltpu.VMEM((EPT, DE), jnp.float32),
            pltpu.VMEM_SHARED((N_PAD, DE), jnp.float32),
        ],
        compiler_params=pltpu.CompilerParams(use_tc_tiling_on_sc=False),
    )(idx3, attr_pad, zeros_n)


def _mlp_kernel(x_ref, ea0_ref, ea1_ref, w1x_ref, w1e_ref, b1_ref,
                w2_ref, b2_ref, w3_ref, b3_ref, out_ref):
    ea = ea0_ref[...] + ea1_ref[...]
    h = (jnp.dot(x_ref[...], w1x_ref[...], preferred_element_type=jnp.float32)
         + jnp.dot(ea, w1e_ref[...], preferred_element_type=jnp.float32)
         + b1_ref[...])
    h = jnp.where(h >= 0, h, 0.01 * h)
    h = jnp.dot(h, w2_ref[...], preferred_element_type=jnp.float32) + b2_ref[...]
    h = jnp.where(h >= 0, h, 0.01 * h)
    h = jnp.dot(h, w3_ref[...], preferred_element_type=jnp.float32) + b3_ref[...]
    out_ref[...] = h


def _mlp(x, ea0, ea1, w1x, w1e, b1, w2, b2, w3, b3):
    BLK = 1000
    grid = (N // BLK,)
    full = lambda shape: pl.BlockSpec(shape, lambda i: (0, 0))
    return pl.pallas_call(
        _mlp_kernel,
        grid=grid,
        in_specs=[
            pl.BlockSpec((BLK, D), lambda i: (i, 0)),
            pl.BlockSpec((BLK, DE), lambda i: (i, 0)),
            pl.BlockSpec((BLK, DE), lambda i: (i, 0)),
            full((D, H)),
            full((DE, H)),
            full((1, H)),
            full((H, H)),
            full((1, H)),
            full((H, H)),
            full((1, H)),
        ],
        out_specs=pl.BlockSpec((BLK, H), lambda i: (i, 0)),
        out_shape=jax.ShapeDtypeStruct((N, H), jnp.float32),
    )(x, ea0, ea1, w1x, w1e, b1, w2, b2, w3, b3)


def kernel(x, edge_index, edge_attr, u, batch, W1, b1, W2, b2, W3, b3):
    row = edge_index[0]
    pad = E_PAD - E
    row_pad = jnp.concatenate([row, jnp.zeros((pad,), jnp.int32)])
    attr_pad = jnp.concatenate(
        [edge_attr, jnp.zeros((pad, DE), jnp.float32)], axis=0)
    idx3 = row_pad.reshape(NW, KCH, CHUNK)
    zeros_n = jnp.zeros((N_PAD, DE), jnp.float32)

    ea2 = _sc_scatter(idx3, attr_pad, zeros_n)

    return _mlp(x, ea2[0, :N], ea2[1, :N], W1[:D], W1[D:], b1[None, :],
                W2, b2[None, :], W3, b3[None, :])


# speedup vs baseline: 4.4770x; 1.4698x over previous
import functools

import jax
import jax.numpy as jnp
from jax import lax
from jax.experimental import pallas as pl
from jax.experimental.pallas import tpu as pltpu
from jax.experimental.pallas import tpu_sc as plsc

N = 10000
E = 160000
D = 256
DE = 16
H = 256

NC = 2
NS = 16
NW = NC * NS
CHUNK = 128
KCH = -(-E // (NW * CHUNK))
EPT = KCH * CHUNK
EPE = E // NW

N_PAD = 10240
ROWS_PER_TILE = N_PAD // NS


def _sc_scatter_kernel(idx_hbm, attr_hbm, zero_hbm, out_hbm, idx_v, attr_v, acc):
    c = lax.axis_index("c")
    s = lax.axis_index("s")
    wid = c * NS + s

    pltpu.sync_copy(zero_hbm.at[pl.ds(s * ROWS_PER_TILE, ROWS_PER_TILE)],
                    acc.at[pl.ds(s * ROWS_PER_TILE, ROWS_PER_TILE)])

    pltpu.sync_copy(idx_hbm.at[wid], idx_v)
    pltpu.sync_copy(attr_hbm.at[pl.ds(wid * EPE, EPE)],
                    attr_v.at[pl.ds(0, EPE)])

    plsc.subcore_barrier()

    @pl.loop(0, KCH)
    def _scatter(j):
        pltpu.sync_copy(attr_v.at[pl.ds(j * CHUNK, CHUNK)],
                        acc.at[idx_v.at[j]], add=True)

    plsc.subcore_barrier()

    pltpu.sync_copy(acc.at[pl.ds(s * ROWS_PER_TILE, ROWS_PER_TILE)],
                    out_hbm.at[c, pl.ds(s * ROWS_PER_TILE, ROWS_PER_TILE)])


def _sc_scatter(idx3, attr_pad, zeros_n):
    mesh = plsc.VectorSubcoreMesh(core_axis_name="c", subcore_axis_name="s")
    return pl.kernel(
        _sc_scatter_kernel,
        out_type=jax.ShapeDtypeStruct((NC, N_PAD, DE), jnp.float32),
        mesh=mesh,
        scratch_types=[
            pltpu.VMEM((KCH, CHUNK), jnp.int32),
            pltpu.VMEM((EPT, DE), jnp.float32),
            pltpu.VMEM_SHARED((N_PAD, DE), jnp.float32),
        ],
        compiler_params=pltpu.CompilerParams(use_tc_tiling_on_sc=False),
    )(idx3, attr_pad, zeros_n)


def _mlp_kernel(x_ref, ea0_ref, ea1_ref, w1x_ref, w1e_ref, b1_ref,
                w2_ref, b2_ref, w3_ref, b3_ref, out_ref):
    ea = ea0_ref[0] + ea1_ref[0]
    h = (jnp.dot(x_ref[...], w1x_ref[...], preferred_element_type=jnp.float32)
         + jnp.dot(ea, w1e_ref[...], preferred_element_type=jnp.float32)
         + b1_ref[...])
    h = jnp.where(h >= 0, h, 0.01 * h)
    h = jnp.dot(h, w2_ref[...], preferred_element_type=jnp.float32) + b2_ref[...]
    h = jnp.where(h >= 0, h, 0.01 * h)
    h = jnp.dot(h, w3_ref[...], preferred_element_type=jnp.float32) + b3_ref[...]
    out_ref[...] = h


def _mlp(x, ea2, w1x, w1e, b1, w2, b2, w3, b3):
    BLK = 1000
    grid = (N // BLK,)
    full = lambda shape: pl.BlockSpec(shape, lambda i: (0, 0))
    return pl.pallas_call(
        _mlp_kernel,
        grid=grid,
        in_specs=[
            pl.BlockSpec((BLK, D), lambda i: (i, 0)),
            pl.BlockSpec((1, BLK, DE), lambda i: (0, i, 0)),
            pl.BlockSpec((1, BLK, DE), lambda i: (1, i, 0)),
            full((D, H)),
            full((DE, H)),
            full((1, H)),
            full((H, H)),
            full((1, H)),
            full((H, H)),
            full((1, H)),
        ],
        out_specs=pl.BlockSpec((BLK, H), lambda i: (i, 0)),
        out_shape=jax.ShapeDtypeStruct((N, H), jnp.float32),
    )(x, ea2, ea2, w1x, w1e, b1, w2, b2, w3, b3)


def kernel(x, edge_index, edge_attr, u, batch, W1, b1, W2, b2, W3, b3):
    idx3 = jnp.pad(edge_index[0].reshape(NW, EPE), ((0, 0), (0, EPT - EPE)),
                   constant_values=N).reshape(NW, KCH, CHUNK)
    zeros_n = jnp.zeros((N_PAD, DE), jnp.float32)

    ea2 = _sc_scatter(idx3, edge_attr, zeros_n)

    return _mlp(x, ea2, W1[:D], W1[D:], b1[None, :],
                W2, b2[None, :], W3, b3[None, :])


# speedup vs baseline: 4.4894x; 1.0028x over previous
import functools

import jax
import jax.numpy as jnp
from jax import lax
from jax.experimental import pallas as pl
from jax.experimental.pallas import tpu as pltpu
from jax.experimental.pallas import tpu_sc as plsc

N = 10000
E = 160000
D = 256
DE = 16
H = 256

NC = 2
NS = 16
NW = NC * NS
CHUNK = 128
EPT = E // NW
KCH = EPT // CHUNK
TAIL = EPT - KCH * CHUNK

N_PAD = 10240
ROWS_PER_TILE = N_PAD // NS


def _sc_scatter_kernel(idx_hbm, attr_hbm, out_hbm, idx_v, attr_v, zbuf, acc):
    c = lax.axis_index("c")
    s = lax.axis_index("s")
    wid = c * NS + s

    @pl.loop(0, ROWS_PER_TILE)
    def _zero(i):
        zbuf[i] = jnp.zeros((DE,), jnp.float32)

    pltpu.sync_copy(zbuf, acc.at[pl.ds(s * ROWS_PER_TILE, ROWS_PER_TILE)])

    pltpu.sync_copy(idx_hbm.at[pl.ds(wid * EPT, EPT)], idx_v)
    pltpu.sync_copy(attr_hbm.at[pl.ds(wid * EPT, EPT)], attr_v)

    plsc.subcore_barrier()

    @pl.loop(0, KCH)
    def _scatter(j):
        pltpu.sync_copy(attr_v.at[pl.ds(j * CHUNK, CHUNK)],
                        acc.at[idx_v.at[pl.ds(j * CHUNK, CHUNK)]], add=True)

    pltpu.sync_copy(attr_v.at[pl.ds(KCH * CHUNK, TAIL)],
                    acc.at[idx_v.at[pl.ds(KCH * CHUNK, TAIL)]], add=True)

    plsc.subcore_barrier()

    pltpu.sync_copy(acc.at[pl.ds(s * ROWS_PER_TILE, ROWS_PER_TILE)],
                    out_hbm.at[c, pl.ds(s * ROWS_PER_TILE, ROWS_PER_TILE)])


def _sc_scatter(row, attr):
    mesh = plsc.VectorSubcoreMesh(core_axis_name="c", subcore_axis_name="s")
    return pl.kernel(
        _sc_scatter_kernel,
        out_type=jax.ShapeDtypeStruct((NC, N_PAD, DE), jnp.float32),
        mesh=mesh,
        scratch_types=[
            pltpu.VMEM((EPT,), jnp.int32),
            pltpu.VMEM((EPT, DE), jnp.float32),
            pltpu.VMEM((ROWS_PER_TILE, DE), jnp.float32),
            pltpu.VMEM_SHARED((N_PAD, DE), jnp.float32),
        ],
        compiler_params=pltpu.CompilerParams(use_tc_tiling_on_sc=False),
    )(row, attr)


def _mlp_kernel(x_ref, ea0_ref, ea1_ref, w1x_ref, w1e_ref, b1_ref,
                w2_ref, b2_ref, w3_ref, b3_ref, out_ref):
    ea = ea0_ref[0] + ea1_ref[0]
    h = (jnp.dot(x_ref[...], w1x_ref[...], preferred_element_type=jnp.float32)
         + jnp.dot(ea, w1e_ref[...], preferred_element_type=jnp.float32)
         + b1_ref[...])
    h = jnp.where(h >= 0, h, 0.01 * h)
    h = jnp.dot(h, w2_ref[...], preferred_element_type=jnp.float32) + b2_ref[...]
    h = jnp.where(h >= 0, h, 0.01 * h)
    h = jnp.dot(h, w3_ref[...], preferred_element_type=jnp.float32) + b3_ref[...]
    out_ref[...] = h


def _mlp(x, ea2, w1x, w1e, b1, w2, b2, w3, b3):
    BLK = 1000
    grid = (N // BLK,)
    full = lambda shape: pl.BlockSpec(shape, lambda i: (0, 0))
    return pl.pallas_call(
        _mlp_kernel,
        grid=grid,
        in_specs=[
            pl.BlockSpec((BLK, D), lambda i: (i, 0)),
            pl.BlockSpec((1, BLK, DE), lambda i: (0, i, 0)),
            pl.BlockSpec((1, BLK, DE), lambda i: (1, i, 0)),
            full((D, H)),
            full((DE, H)),
            full((1, H)),
            full((H, H)),
            full((1, H)),
            full((H, H)),
            full((1, H)),
        ],
        out_specs=pl.BlockSpec((BLK, H), lambda i: (i, 0)),
        out_shape=jax.ShapeDtypeStruct((N, H), jnp.float32),
    )(x, ea2, ea2, w1x, w1e, b1, w2, b2, w3, b3)


def kernel(x, edge_index, edge_attr, u, batch, W1, b1, W2, b2, W3, b3):
    ea2 = _sc_scatter(edge_index[0], edge_attr)

    return _mlp(x, ea2, W1[:D], W1[D:], b1[None, :],
                W2, b2[None, :], W3, b3[None, :])
